# baseline (speedup 1.0000x reference)
import functools

import jax
import jax.numpy as jnp
from jax import lax
from jax.experimental import pallas as pl
from jax.experimental.pallas import tpu as pltpu
from jax.experimental.pallas import tpu_sc as plsc

H = W = 384
HW = H * W
CIN, NC = 96, 19
C0 = CIN + NC
O0 = C0 // 2
O1 = O0 // 2
GATE = 0.4
NEG = -3.0e38

BH = 16
NB = H // BH
PB = NB + 2
BL = BH * W
PHW = PB * BL


RPW = H // 32


def _sc_mask_body(c_hbm, m_hbm, cbuf, ubuf, sem, sem2):
    wid = lax.axis_index("s") * 2 + lax.axis_index("c")
    r0 = wid * RPW
    win = (RPW + 2) * W
    nv = win // 16

    descs = [pltpu.async_copy(
        c_hbm.at[pl.ds(c * HW + r0 * W, RPW * W)],
        cbuf.at[pl.ds(c * win + W, RPW * W)], sem)
        for c in range(NC)]

    @pl.when(wid > 0)
    def _():
        hd = [pltpu.async_copy(
            c_hbm.at[pl.ds(c * HW + (r0 - 1) * W, W)],
            cbuf.at[pl.ds(c * win, W)], sem2)
            for c in range(NC)]
        for d in hd:
            d.wait()

    @pl.when(wid < 31)
    def _():
        hd = [pltpu.async_copy(
            c_hbm.at[pl.ds(c * HW + (r0 + RPW) * W, W)],
            cbuf.at[pl.ds(c * win + (RPW + 1) * W, W)], sem2)
            for c in range(NC)]
        for d in hd:
            d.wait()

    for d in descs:
        d.wait()

    def chmax(j, _):
        off = j * 16
        m = cbuf[pl.ds(off, 16)]
        for c in range(1, NC):
            m = jnp.maximum(m, cbuf[pl.ds(c * win + off, 16)])
        ubuf[pl.ds(off, 16)] = m
        return 0

    lax.fori_loop(0, nv, chmax, 0)

    posv = jnp.full((16,), -NEG, jnp.float32)
    @pl.when(wid == 0)
    def _():
        for j in range(W // 16):
            ubuf[pl.ds(j * 16, 16)] = posv

    @pl.when(wid == 31)
    def _():
        for j in range(W // 16):
            ubuf[pl.ds((RPW + 1) * W + j * 16, 16)] = posv

    def vmax(k, _):
        a = ubuf[pl.ds(k * 16, 16)]
        b = ubuf[pl.ds(W + k * 16, 16)]
        c = ubuf[pl.ds(2 * W + k * 16, 16)]
        ubuf[pl.ds(k * 16, 16)] = 1.0 - jnp.minimum(jnp.minimum(a, b), c)
        return 0

    lax.fori_loop(0, RPW * (W // 16), vmax, 0)
    pltpu.sync_copy(ubuf.at[pl.ds(0, RPW * W)], m_hbm.at[pl.ds(r0 * W, RPW * W)])


def _compute_mask_sc(coarse2):
    mesh = plsc.VectorSubcoreMesh(core_axis_name="c", subcore_axis_name="s")
    k = pl.kernel(
        _sc_mask_body,
        mesh=mesh,
        out_type=jax.ShapeDtypeStruct((HW,), jnp.float32),
        scratch_types=[
            pltpu.VMEM((NC * (RPW + 2) * W,), jnp.float32),
            pltpu.VMEM(((RPW + 2) * W,), jnp.float32),
            pltpu.SemaphoreType.DMA,
            pltpu.SemaphoreType.DMA,
        ],
    )
    return k(coarse2.reshape(-1)).reshape(H, W)


def _mlp_chunk(xc, ws):
    win, bin_, wm0, bm0, wm1, bm1, wout, bout = ws
    h = jnp.clip(jnp.dot(win, xc, preferred_element_type=jnp.float32) + bin_, 0.0, 6.0)
    h = h + jnp.clip(jnp.dot(wm0, h, preferred_element_type=jnp.float32) + bm0, 0.0, 6.0)
    h = h + jnp.clip(jnp.dot(wm1, h, preferred_element_type=jnp.float32) + bm1, 0.0, 6.0)
    return jnp.clip(jnp.dot(wout, h, preferred_element_type=jnp.float32) + bout, 0.0, 6.0)


def _mlp_body(n_in, m_ref, *refs):
    in_refs = refs[:n_in]
    w_refs = refs[n_in:n_in + 8]
    out_ref = refs[n_in + 8]
    ws = tuple(r[...] for r in w_refs)
    i = pl.program_id(0)
    interior = jnp.logical_and(i > 0, i < PB - 1)

    for row in range(BH):
        sl = slice(row * W, (row + 1) * W)
        parts = [r[:, sl] for r in in_refs]
        xc = parts[0] if n_in == 1 else jnp.concatenate(parts, axis=0)
        rm = m_ref[:, row, :]
        npad = jnp.full((1, 1), NEG, jnp.float32)
        lf = jnp.concatenate([rm[:, 1:], npad], axis=1)
        rt = jnp.concatenate([npad, rm[:, :-1]], axis=1)
        mrow = jnp.maximum(jnp.maximum(rm, lf), rt)
        act = jnp.logical_and(interior, jnp.max(mrow) > GATE)

        @pl.when(act)
        def _(xc=xc, mrow=mrow, sl=sl):
            ur = _mlp_chunk(xc.astype(jnp.float32), ws)
            out_ref[:, sl] = jnp.where(
                mrow > GATE, ur, xc.astype(jnp.float32)).astype(out_ref.dtype)

        @pl.when(jnp.logical_not(act))
        def _(xc=xc, sl=sl):
            out_ref[:, sl] = jnp.where(
                interior, xc, 0).astype(out_ref.dtype)


def _run_mlp(in_arrays, in_padded, mask, wlist, cout, out_dtype):
    n_in = len(in_arrays)
    if in_padded:
        in_specs = [pl.BlockSpec((a.shape[0], BL), lambda i: (0, i))
                    for a in in_arrays]
    else:
        in_specs = [pl.BlockSpec((a.shape[0], BL),
                                 lambda i: (0, jnp.clip(i - 1, 0, NB - 1)))
                    for a in in_arrays]
    mask3 = mask.reshape(NB, BH, W)
    m_spec = pl.BlockSpec((1, BH, W), lambda i: (jnp.clip(i - 1, 0, NB - 1), 0, 0))
    w_specs = [pl.BlockSpec(w.shape, lambda i: (0, 0)) for w in wlist]
    return pl.pallas_call(
        functools.partial(_mlp_body, n_in),
        grid=(PB,),
        in_specs=[m_spec] + in_specs + w_specs,
        out_specs=pl.BlockSpec((cout, BL), lambda i: (0, i)),
        out_shape=jax.ShapeDtypeStruct((cout, PHW), out_dtype),
    )(mask3, *in_arrays, *wlist)


def _sep_flat(v, dw, pw, be):
    Cc, Lv = v.shape
    Lo = Lv - 2 * W
    lane = lax.broadcasted_iota(jnp.int32, (1, Lv), 1)
    bm0 = jnp.where(lane % W == 0, 0.0, 1.0).astype(jnp.bfloat16)
    bm1 = jnp.where(lane % W == W - 1, 0.0, 1.0).astype(jnp.bfloat16)
    vm = pltpu.roll(v, 1, 1) * bm0
    vp = pltpu.roll(v, Lv - 1, 1) * bm1
    acc = None
    for dh in range(3):
        o = dh * W
        t = (vm[:, o:o + Lo] * dw[:, dh, 0:1]
             + v[:, o:o + Lo] * dw[:, dh, 1:2]
             + vp[:, o:o + Lo] * dw[:, dh, 2:3])
        acc = t if acc is None else acc + t
    y = jnp.dot(pw, acc, preferred_element_type=jnp.float32) + be
    return jnp.maximum(y, 0.0)


def _conv_body(xp_ref, xc_ref, xn_ref,
               dw0_ref, pw0_ref, be0_ref,
               dw1_ref, pw1_ref, be1_ref,
               dwo_ref, pwo_ref, beo_ref, out_ref):
    i = pl.program_id(0)
    edge = jnp.logical_or(i == 0, i == NB - 1)
    xs = jnp.concatenate(
        [xp_ref[:, 13 * W:], xc_ref[...], xn_ref[:, :3 * W]], axis=1)
    prow = lax.broadcasted_iota(jnp.int32, (1, 22 * W), 1) // W + (i * BH + 13)
    vb = jnp.where(
        jnp.logical_and(prow >= BH, prow < BH * (NB + 1)),
        1.0, 0.0).astype(jnp.bfloat16)

    def zeropad(t, sl):
        return lax.cond(edge, lambda a: a * vb[:, sl], lambda a: a, t)

    xsb = xs if xs.dtype == jnp.bfloat16 else xs.astype(jnp.bfloat16)
    y1 = _sep_flat(xsb, dw0_ref[...], pw0_ref[...], be0_ref[...])
    if xs.dtype == jnp.bfloat16:
        t1 = xs[:, W:21 * W] + y1.astype(jnp.bfloat16)
    else:
        t1 = (xs[:, W:21 * W] + y1).astype(jnp.bfloat16)
    t1 = zeropad(t1, slice(W, 21 * W))
    y2 = _sep_flat(t1, dw1_ref[...], pw1_ref[...], be1_ref[...])
    t2 = t1[:, W:19 * W] + y2.astype(jnp.bfloat16)
    t2 = zeropad(t2, slice(2 * W, 20 * W))
    yo = _sep_flat(t2, dwo_ref[...], pwo_ref[...], beo_ref[...])
    out_ref[...] = yo.astype(out_ref.dtype)


def _run_convs(xp, cw, cout, out_padded):
    C = xp.shape[0]
    xspec = lambda f: pl.BlockSpec((C, BL), f)
    in_specs = [xspec(lambda i: (0, i)),
                xspec(lambda i: (0, i + 1)),
                xspec(lambda i: (0, i + 2))]
    for w in cw:
        in_specs.append(pl.BlockSpec(w.shape, lambda i, n=w.ndim: (0,) * n))
    if out_padded:
        out_spec = pl.BlockSpec((cout, BL), lambda i: (0, i + 1))
        out_shape = jax.ShapeDtypeStruct((cout, PHW), jnp.bfloat16)
    else:
        out_spec = pl.BlockSpec((cout, BL), lambda i: (0, i))
        out_shape = jax.ShapeDtypeStruct((cout, HW), jnp.float32)
    return pl.pallas_call(
        _conv_body,
        grid=(NB,),
        in_specs=in_specs,
        out_specs=out_spec,
        out_shape=out_shape,
    )(xp, xp, xp, *cw)


def _block_weights(p, b):
    wlist = [p[f'b{b}_win'], p[f'b{b}_bin'].reshape(-1, 1),
             p[f'b{b}_wm0'], p[f'b{b}_bm0'].reshape(-1, 1),
             p[f'b{b}_wm1'], p[f'b{b}_bm1'].reshape(-1, 1),
             p[f'b{b}_wout'], p[f'b{b}_bout'].reshape(-1, 1)]
    scale = 1.0 / jnp.sqrt(1.0 + 1e-5)
    cw = []
    for tag in ('0', '1', 'o'):
        dw = p[f'b{b}_dw{tag}'][:, 0].astype(jnp.bfloat16)
        pw = p[f'b{b}_pw{tag}'][:, :, 0, 0]
        g = p[f'b{b}_g{tag}'] * scale
        pw_eff = (pw * g[:, None]).astype(jnp.bfloat16)
        be = p[f'b{b}_be{tag}'].reshape(-1, 1)
        cw += [dw, pw_eff, be]
    return wlist, cw


def kernel(feature_map, coarse_pred, params):
    fm = feature_map[0].reshape(CIN, HW)
    cp = coarse_pred[0].reshape(NC, HW)
    mask = _compute_mask_sc(cp)

    w0, c0 = _block_weights(params, 0)
    w1, c1 = _block_weights(params, 1)

    x0 = _run_mlp([fm, cp], False, mask, w0, C0, jnp.float32)
    y0 = _run_convs(x0, c0, O0, out_padded=True)
    x1 = _run_mlp([y0], True, mask, w1, O0, jnp.bfloat16)
    y1 = _run_convs(x1, c1, O1, out_padded=False)
    return y1.reshape(1, O1, H, W)


# speedup vs baseline: 1.1064x; 1.1064x over previous
import functools

import jax
import jax.numpy as jnp
from jax import lax
from jax.experimental import pallas as pl
from jax.experimental.pallas import tpu as pltpu
from jax.experimental.pallas import tpu_sc as plsc

H = W = 384
HW = H * W
CIN, NC = 96, 19
C0 = CIN + NC
O0 = C0 // 2
O1 = O0 // 2
GATE = 0.4
NEG = -3.0e38

BH = 32
NB = H // BH
PB = NB + 2
BL = BH * W
PHW = PB * BL


RPW = H // 32


def _sc_mask_body(c_hbm, m_hbm, cbuf, ubuf, sem, sem2):
    wid = lax.axis_index("s") * 2 + lax.axis_index("c")
    r0 = wid * RPW
    win = (RPW + 2) * W
    nv = win // 16

    descs = [pltpu.async_copy(
        c_hbm.at[pl.ds(c * HW + r0 * W, RPW * W)],
        cbuf.at[pl.ds(c * win + W, RPW * W)], sem)
        for c in range(NC)]

    @pl.when(wid > 0)
    def _():
        hd = [pltpu.async_copy(
            c_hbm.at[pl.ds(c * HW + (r0 - 1) * W, W)],
            cbuf.at[pl.ds(c * win, W)], sem2)
            for c in range(NC)]
        for d in hd:
            d.wait()

    @pl.when(wid < 31)
    def _():
        hd = [pltpu.async_copy(
            c_hbm.at[pl.ds(c * HW + (r0 + RPW) * W, W)],
            cbuf.at[pl.ds(c * win + (RPW + 1) * W, W)], sem2)
            for c in range(NC)]
        for d in hd:
            d.wait()

    for d in descs:
        d.wait()

    def chmax(j, _):
        off = j * 16
        m = cbuf[pl.ds(off, 16)]
        for c in range(1, NC):
            m = jnp.maximum(m, cbuf[pl.ds(c * win + off, 16)])
        ubuf[pl.ds(off, 16)] = m
        return 0

    lax.fori_loop(0, nv, chmax, 0)

    posv = jnp.full((16,), -NEG, jnp.float32)
    @pl.when(wid == 0)
    def _():
        for j in range(W // 16):
            ubuf[pl.ds(j * 16, 16)] = posv

    @pl.when(wid == 31)
    def _():
        for j in range(W // 16):
            ubuf[pl.ds((RPW + 1) * W + j * 16, 16)] = posv

    def vmax(k, _):
        a = ubuf[pl.ds(k * 16, 16)]
        b = ubuf[pl.ds(W + k * 16, 16)]
        c = ubuf[pl.ds(2 * W + k * 16, 16)]
        ubuf[pl.ds(k * 16, 16)] = 1.0 - jnp.minimum(jnp.minimum(a, b), c)
        return 0

    lax.fori_loop(0, RPW * (W // 16), vmax, 0)
    pltpu.sync_copy(ubuf.at[pl.ds(0, RPW * W)], m_hbm.at[pl.ds(r0 * W, RPW * W)])


def _compute_mask_sc(coarse2):
    mesh = plsc.VectorSubcoreMesh(core_axis_name="c", subcore_axis_name="s")
    k = pl.kernel(
        _sc_mask_body,
        mesh=mesh,
        out_type=jax.ShapeDtypeStruct((HW,), jnp.float32),
        scratch_types=[
            pltpu.VMEM((NC * (RPW + 2) * W,), jnp.float32),
            pltpu.VMEM(((RPW + 2) * W,), jnp.float32),
            pltpu.SemaphoreType.DMA,
            pltpu.SemaphoreType.DMA,
        ],
    )
    return k(coarse2.reshape(-1)).reshape(H, W)


def _mlp_chunk(xc, ws):
    win, bin_, wm0, bm0, wm1, bm1, wout, bout = ws
    h = jnp.clip(jnp.dot(win, xc, preferred_element_type=jnp.float32) + bin_, 0.0, 6.0)
    h = h + jnp.clip(jnp.dot(wm0, h, preferred_element_type=jnp.float32) + bm0, 0.0, 6.0)
    h = h + jnp.clip(jnp.dot(wm1, h, preferred_element_type=jnp.float32) + bm1, 0.0, 6.0)
    return jnp.clip(jnp.dot(wout, h, preferred_element_type=jnp.float32) + bout, 0.0, 6.0)


def _mlp_body(n_in, m_ref, *refs):
    in_refs = refs[:n_in]
    w_refs = refs[n_in:n_in + 8]
    out_ref = refs[n_in + 8]
    ws = tuple(r[...] for r in w_refs)
    i = pl.program_id(0)
    interior = jnp.logical_and(i > 0, i < PB - 1)

    for row in range(BH):
        sl = slice(row * W, (row + 1) * W)
        parts = [r[:, sl] for r in in_refs]
        xc = parts[0] if n_in == 1 else jnp.concatenate(parts, axis=0)
        rm = m_ref[:, row, :]
        npad = jnp.full((1, 1), NEG, jnp.float32)
        lf = jnp.concatenate([rm[:, 1:], npad], axis=1)
        rt = jnp.concatenate([npad, rm[:, :-1]], axis=1)
        mrow = jnp.maximum(jnp.maximum(rm, lf), rt)
        act = jnp.logical_and(interior, jnp.max(mrow) > GATE)

        @pl.when(act)
        def _(xc=xc, mrow=mrow, sl=sl):
            ur = _mlp_chunk(xc.astype(jnp.float32), ws)
            out_ref[:, sl] = jnp.where(
                mrow > GATE, ur, xc.astype(jnp.float32)).astype(out_ref.dtype)

        @pl.when(jnp.logical_not(act))
        def _(xc=xc, sl=sl):
            out_ref[:, sl] = jnp.where(
                interior, xc, 0).astype(out_ref.dtype)


def _run_mlp(in_arrays, in_padded, mask, wlist, cout, out_dtype):
    n_in = len(in_arrays)
    if in_padded:
        in_specs = [pl.BlockSpec((a.shape[0], BL), lambda i: (0, i))
                    for a in in_arrays]
    else:
        in_specs = [pl.BlockSpec((a.shape[0], BL),
                                 lambda i: (0, jnp.clip(i - 1, 0, NB - 1)))
                    for a in in_arrays]
    mask3 = mask.reshape(NB, BH, W)
    m_spec = pl.BlockSpec((1, BH, W), lambda i: (jnp.clip(i - 1, 0, NB - 1), 0, 0))
    w_specs = [pl.BlockSpec(w.shape, lambda i: (0, 0)) for w in wlist]
    return pl.pallas_call(
        functools.partial(_mlp_body, n_in),
        grid=(PB,),
        in_specs=[m_spec] + in_specs + w_specs,
        out_specs=pl.BlockSpec((cout, BL), lambda i: (0, i)),
        out_shape=jax.ShapeDtypeStruct((cout, PHW), out_dtype),
    )(mask3, *in_arrays, *wlist)


def _sep_flat(v, dw, pw, be):
    Cc, Lv = v.shape
    Lo = Lv - 2 * W
    lane = lax.broadcasted_iota(jnp.int32, (1, Lv), 1)
    bm0 = jnp.where(lane % W == 0, 0.0, 1.0).astype(jnp.bfloat16)
    bm1 = jnp.where(lane % W == W - 1, 0.0, 1.0).astype(jnp.bfloat16)
    vm = pltpu.roll(v, 1, 1) * bm0
    vp = pltpu.roll(v, Lv - 1, 1) * bm1
    acc = None
    for dh in range(3):
        o = dh * W
        t = (vm[:, o:o + Lo] * dw[:, dh, 0:1]
             + v[:, o:o + Lo] * dw[:, dh, 1:2]
             + vp[:, o:o + Lo] * dw[:, dh, 2:3])
        acc = t if acc is None else acc + t
    y = jnp.dot(pw, acc, preferred_element_type=jnp.float32) + be
    return jnp.maximum(y, 0.0)


def _conv_body(xp_ref, xc_ref, xn_ref,
               dw0_ref, pw0_ref, be0_ref,
               dw1_ref, pw1_ref, be1_ref,
               dwo_ref, pwo_ref, beo_ref, out_ref):
    i = pl.program_id(0)
    edge = jnp.logical_or(i == 0, i == NB - 1)
    xs = jnp.concatenate(
        [xp_ref[:, (BH - 3) * W:], xc_ref[...], xn_ref[:, :3 * W]], axis=1)
    prow = (lax.broadcasted_iota(jnp.int32, (1, (BH + 6) * W), 1) // W
            + (i * BH + BH - 3))
    vb = jnp.where(
        jnp.logical_and(prow >= BH, prow < BH * (NB + 1)),
        1.0, 0.0).astype(jnp.bfloat16)

    def zeropad(t, sl):
        return lax.cond(edge, lambda a: a * vb[:, sl], lambda a: a, t)

    xsb = xs if xs.dtype == jnp.bfloat16 else xs.astype(jnp.bfloat16)
    y1 = _sep_flat(xsb, dw0_ref[...], pw0_ref[...], be0_ref[...])
    if xs.dtype == jnp.bfloat16:
        t1 = xs[:, W:(BH + 5) * W] + y1.astype(jnp.bfloat16)
    else:
        t1 = (xs[:, W:(BH + 5) * W] + y1).astype(jnp.bfloat16)
    t1 = zeropad(t1, slice(W, (BH + 5) * W))
    y2 = _sep_flat(t1, dw1_ref[...], pw1_ref[...], be1_ref[...])
    t2 = t1[:, W:(BH + 3) * W] + y2.astype(jnp.bfloat16)
    t2 = zeropad(t2, slice(2 * W, (BH + 4) * W))
    yo = _sep_flat(t2, dwo_ref[...], pwo_ref[...], beo_ref[...])
    out_ref[...] = yo.astype(out_ref.dtype)


def _run_convs(xp, cw, cout, out_padded):
    C = xp.shape[0]
    xspec = lambda f: pl.BlockSpec((C, BL), f)
    in_specs = [xspec(lambda i: (0, i)),
                xspec(lambda i: (0, i + 1)),
                xspec(lambda i: (0, i + 2))]
    for w in cw:
        in_specs.append(pl.BlockSpec(w.shape, lambda i, n=w.ndim: (0,) * n))
    if out_padded:
        out_spec = pl.BlockSpec((cout, BL), lambda i: (0, i + 1))
        out_shape = jax.ShapeDtypeStruct((cout, PHW), jnp.bfloat16)
    else:
        out_spec = pl.BlockSpec((cout, BL), lambda i: (0, i))
        out_shape = jax.ShapeDtypeStruct((cout, HW), jnp.float32)
    return pl.pallas_call(
        _conv_body,
        grid=(NB,),
        in_specs=in_specs,
        out_specs=out_spec,
        out_shape=out_shape,
    )(xp, xp, xp, *cw)


def _block_weights(p, b):
    wlist = [p[f'b{b}_win'], p[f'b{b}_bin'].reshape(-1, 1),
             p[f'b{b}_wm0'], p[f'b{b}_bm0'].reshape(-1, 1),
             p[f'b{b}_wm1'], p[f'b{b}_bm1'].reshape(-1, 1),
             p[f'b{b}_wout'], p[f'b{b}_bout'].reshape(-1, 1)]
    scale = 1.0 / jnp.sqrt(1.0 + 1e-5)
    cw = []
    for tag in ('0', '1', 'o'):
        dw = p[f'b{b}_dw{tag}'][:, 0].astype(jnp.bfloat16)
        pw = p[f'b{b}_pw{tag}'][:, :, 0, 0]
        g = p[f'b{b}_g{tag}'] * scale
        pw_eff = (pw * g[:, None]).astype(jnp.bfloat16)
        be = p[f'b{b}_be{tag}'].reshape(-1, 1)
        cw += [dw, pw_eff, be]
    return wlist, cw


def kernel(feature_map, coarse_pred, params):
    fm = feature_map[0].reshape(CIN, HW)
    cp = coarse_pred[0].reshape(NC, HW)
    mask = _compute_mask_sc(cp)

    w0, c0 = _block_weights(params, 0)
    w1, c1 = _block_weights(params, 1)

    x0 = _run_mlp([fm, cp], False, mask, w0, C0, jnp.float32)
    y0 = _run_convs(x0, c0, O0, out_padded=True)
    x1 = _run_mlp([y0], True, mask, w1, O0, jnp.bfloat16)
    y1 = _run_convs(x1, c1, O1, out_padded=False)
    return y1.reshape(1, O1, H, W)
